# baseline (speedup 1.0000x reference)
import jax
import jax.numpy as jnp
from jax.experimental import pallas as pl
from jax.experimental.pallas import tpu as pltpu

_S = 2


def _malconv_kernel(zt_ref, w1_ref, w2_ref, b1_ref, b2_ref, out_ref):
    zbt = zt_ref[0]
    e, tc = zbt.shape
    nw = tc // 512
    zz = zbt.astype(jnp.bfloat16).reshape(e, nw, 512).transpose(1, 0, 2).reshape(nw, 512 * e)
    c1 = jnp.dot(zz, w1_ref[...], preferred_element_type=jnp.float32) + b1_ref[...]
    c2 = jnp.dot(zz, w2_ref[...], preferred_element_type=jnp.float32) + b2_ref[...]
    g = c1 * jax.nn.sigmoid(c2)
    out_ref[0, 0] = jnp.max(g, axis=0, keepdims=True)


def kernel(z, W1, b1, W2, b2):
    B, T, E = z.shape
    C, _, K = W1.shape
    KE = K * E
    zt = jnp.transpose(z, (0, 2, 1))
    W1t = W1.transpose(1, 2, 0).reshape(KE, C).astype(jnp.bfloat16)
    W2t = W2.transpose(1, 2, 0).reshape(KE, C).astype(jnp.bfloat16)
    out = pl.pallas_call(
        _malconv_kernel,
        grid=(B, _S),
        in_specs=[
            pl.BlockSpec((1, E, T // _S), lambda b, s: (b, 0, s)),
            pl.BlockSpec((KE, C), lambda b, s: (0, 0)),
            pl.BlockSpec((KE, C), lambda b, s: (0, 0)),
            pl.BlockSpec((1, C), lambda b, s: (0, 0)),
            pl.BlockSpec((1, C), lambda b, s: (0, 0)),
        ],
        out_specs=pl.BlockSpec((1, 1, 1, C), lambda b, s: (b, s, 0, 0)),
        out_shape=jax.ShapeDtypeStruct((B, _S, 1, C), jnp.float32),
        compiler_params=pltpu.CompilerParams(
            dimension_semantics=("parallel", "parallel"),
        ),
    )(zt, W1t, W2t, b1.reshape(1, C), b2.reshape(1, C))
    return jnp.max(out.reshape(B, _S, C), axis=1)


# speedup vs baseline: 1.1146x; 1.1146x over previous
import jax
import jax.numpy as jnp
from jax.experimental import pallas as pl
from jax.experimental.pallas import tpu as pltpu

_BB = 2


def _malconv_kernel(zt_ref, w1_ref, w2_ref, b1_ref, b2_ref, out_ref):
    e = zt_ref.shape[1]
    tc = zt_ref.shape[2]
    nw = tc // 512
    for i in range(_BB):
        zbt = zt_ref[i]
        zz = zbt.astype(jnp.bfloat16).reshape(e, nw, 512).transpose(1, 0, 2).reshape(nw, 512 * e)
        c1 = jnp.dot(zz, w1_ref[...], preferred_element_type=jnp.float32) + b1_ref[...]
        c2 = jnp.dot(zz, w2_ref[...], preferred_element_type=jnp.float32) + b2_ref[...]
        g = c1 * jax.nn.sigmoid(c2)
        out_ref[i] = jnp.max(g, axis=0, keepdims=True)


def kernel(z, W1, b1, W2, b2):
    B, T, E = z.shape
    C, _, K = W1.shape
    KE = K * E
    zt = jnp.transpose(z, (0, 2, 1))
    W1t = W1.transpose(1, 2, 0).reshape(KE, C).astype(jnp.bfloat16)
    W2t = W2.transpose(1, 2, 0).reshape(KE, C).astype(jnp.bfloat16)
    out = pl.pallas_call(
        _malconv_kernel,
        grid=(B // _BB,),
        in_specs=[
            pl.BlockSpec((_BB, E, T), lambda b: (b, 0, 0)),
            pl.BlockSpec((KE, C), lambda b: (0, 0)),
            pl.BlockSpec((KE, C), lambda b: (0, 0)),
            pl.BlockSpec((1, C), lambda b: (0, 0)),
            pl.BlockSpec((1, C), lambda b: (0, 0)),
        ],
        out_specs=pl.BlockSpec((_BB, 1, C), lambda b: (b, 0, 0)),
        out_shape=jax.ShapeDtypeStruct((B, 1, C), jnp.float32),
        compiler_params=pltpu.CompilerParams(
            dimension_semantics=("parallel",),
        ),
    )(zt, W1t, W2t, b1.reshape(1, C), b2.reshape(1, C))
    return out.reshape(B, C)
